# baseline (speedup 1.0000x reference)
import jax
import jax.numpy as jnp
from jax.experimental import pallas as pl
from jax.experimental.pallas import tpu as pltpu

_B = 16
_NA = 3
_NC = 80
_NO = 5 + _NC
_CS = [192, 384, 768]
_HWS = [64 * 64, 32 * 32, 16 * 16]
_TOTAL_ROWS = _NA * sum(_HWS)
_ROW_OFF = [0, _NA * _HWS[0], _NA * (_HWS[0] + _HWS[1])]
_NBUF = 3


def _fused_body(f0, w0r, b0r, f1, w1r, b1r, f2, w2r, b2r, out,
                ib0, ob0, ib1, ob1, ib2, ob2,
                ls0, ss0, ls1, ss1, ls2, ss2):
    scales = [
        (f0, w0r, b0r, ib0, ob0, ls0, ss0, _HWS[0], _ROW_OFF[0]),
        (f1, w1r, b1r, ib1, ob1, ls1, ss1, _HWS[1], _ROW_OFF[1]),
        (f2, w2r, b2r, ib2, ob2, ls2, ss2, _HWS[2], _ROW_OFF[2]),
    ]
    for f, wr, br, ibuf, obuf, lsem, ssem, hw, off in scales:
        rows = _NA * hw
        for b in range(_NBUF):
            pltpu.make_async_copy(f.at[b], ibuf.at[b], lsem.at[b]).start()
        for b in range(_B):
            slot = b % _NBUF
            pltpu.make_async_copy(f.at[b], ibuf.at[slot], lsem.at[slot]).wait()
            if b >= _NBUF:
                pltpu.make_async_copy(
                    obuf.at[slot], out.at[b - _NBUF, pl.ds(off, rows)],
                    ssem.at[slot],
                ).wait()
            x = ibuf[slot].astype(jnp.bfloat16)
            for a in range(_NA):
                w = wr[a].astype(jnp.bfloat16)
                y = jax.lax.dot_general(
                    x, w, (((0,), (1,)), ((), ())),
                    preferred_element_type=jnp.float32,
                )
                obuf[slot, a * hw:(a + 1) * hw, :] = y + br[a][None, :]
            pltpu.make_async_copy(
                obuf.at[slot], out.at[b, pl.ds(off, rows)], ssem.at[slot]
            ).start()
            if b + _NBUF < _B:
                pltpu.make_async_copy(
                    f.at[b + _NBUF], ibuf.at[slot], lsem.at[slot]
                ).start()
    for f, wr, br, ibuf, obuf, lsem, ssem, hw, off in scales:
        rows = _NA * hw
        for b in range(_B - _NBUF, _B):
            slot = b % _NBUF
            pltpu.make_async_copy(
                obuf.at[slot], out.at[b, pl.ds(off, rows)], ssem.at[slot]
            ).wait()


def kernel(features_0, features_1, features_2, W0, b0, W1, b1, W2, b2):
    feats = [features_0.reshape(_B, _CS[0], _HWS[0]),
             features_1.reshape(_B, _CS[1], _HWS[1]),
             features_2.reshape(_B, _CS[2], _HWS[2])]
    ws = [W0.reshape(_NA, _NO, _CS[0]),
          W1.reshape(_NA, _NO, _CS[1]),
          W2.reshape(_NA, _NO, _CS[2])]
    bs = [b0.reshape(_NA, _NO), b1.reshape(_NA, _NO), b2.reshape(_NA, _NO)]

    any_spec = pl.BlockSpec(memory_space=pl.ANY)
    vmem_spec = pl.BlockSpec(memory_space=pltpu.VMEM)
    scratch = []
    for i in range(3):
        scratch.append(pltpu.VMEM((_NBUF, _CS[i], _HWS[i]), jnp.float32))
        scratch.append(pltpu.VMEM((_NBUF, _NA * _HWS[i], _NO), jnp.float32))
    scratch += [pltpu.SemaphoreType.DMA((_NBUF,))] * 6

    return pl.pallas_call(
        _fused_body,
        in_specs=[any_spec, vmem_spec, vmem_spec] * 3,
        out_specs=any_spec,
        out_shape=jax.ShapeDtypeStruct((_B, _TOTAL_ROWS, _NO), jnp.float32),
        scratch_shapes=scratch,
        compiler_params=pltpu.CompilerParams(
            vmem_limit_bytes=100 * 1024 * 1024,
        ),
    )(feats[0], ws[0], bs[0], feats[1], ws[1], bs[1], feats[2], ws[2], bs[2])


# speedup vs baseline: 1.0114x; 1.0114x over previous
import jax
import jax.numpy as jnp
from jax.experimental import pallas as pl
from jax.experimental.pallas import tpu as pltpu

_B = 16
_NA = 3
_NC = 80
_NO = 5 + _NC
_CS = [192, 384, 768]
_HWS = [64 * 64, 32 * 32, 16 * 16]
_TOTAL_ROWS = _NA * sum(_HWS)
_ROW_OFF = [0, _NA * _HWS[0], _NA * (_HWS[0] + _HWS[1])]


def _body(f0, w0, b0, f1, w1, b1, f2, w2, b2, out):
    for f_ref, w_ref, b_ref, hw, off in (
        (f0, w0, b0, _HWS[0], _ROW_OFF[0]),
        (f1, w1, b1, _HWS[1], _ROW_OFF[1]),
        (f2, w2, b2, _HWS[2], _ROW_OFF[2]),
    ):
        f = f_ref[0]
        for a in range(_NA):
            w = w_ref[a]
            acc = jax.lax.dot_general(
                f, w, (((0,), (0,)), ((), ())),
                preferred_element_type=jnp.float32)
            out[0, pl.ds(off + a * hw, hw), :] = acc + b_ref[a]


def kernel(features_0, features_1, features_2, W0, b0, W1, b1, W2, b2):
    feats = [features_0.reshape(_B, _CS[0], _HWS[0]),
             features_1.reshape(_B, _CS[1], _HWS[1]),
             features_2.reshape(_B, _CS[2], _HWS[2])]
    wts = [W0.reshape(_NA, _NO, _CS[0]).transpose(0, 2, 1),
           W1.reshape(_NA, _NO, _CS[1]).transpose(0, 2, 1),
           W2.reshape(_NA, _NO, _CS[2]).transpose(0, 2, 1)]
    bss = [b0.reshape(_NA, 1, _NO), b1.reshape(_NA, 1, _NO),
           b2.reshape(_NA, 1, _NO)]

    in_specs = []
    args = []
    for i in range(3):
        in_specs.append(
            pl.BlockSpec((1, _CS[i], _HWS[i]), lambda b: (b, 0, 0)))
        in_specs.append(
            pl.BlockSpec((_NA, _CS[i], _NO), lambda b: (0, 0, 0)))
        in_specs.append(pl.BlockSpec((_NA, 1, _NO), lambda b: (0, 0, 0)))
        args += [feats[i], wts[i], bss[i]]

    return pl.pallas_call(
        _body,
        grid=(_B,),
        in_specs=in_specs,
        out_specs=pl.BlockSpec((1, _TOTAL_ROWS, _NO), lambda b: (b, 0, 0)),
        out_shape=jax.ShapeDtypeStruct((_B, _TOTAL_ROWS, _NO), jnp.float32),
        compiler_params=pltpu.CompilerParams(
            vmem_limit_bytes=100 * 1024 * 1024,
        ),
    )(*args)
